# baseline (speedup 1.0000x reference)
import functools

import jax
import jax.numpy as jnp
from jax import lax
from jax.experimental import pallas as pl
from jax.experimental.pallas import tpu as pltpu
from jax.experimental.pallas import tpu_sc as plsc

N_U = 10000
N_V = 10000
N_E = 320000
D_N = 128
D_E = 16
EPS = 1e-5

NC = 2
NS = 16
NW = NC * NS
RPT = 320
NPAD = NW * RPT
TRASH = RPT
ACC_R = RPT + 8
CHUNK = 1280
NCHUNKS = N_E // CHUNK
GRPS = CHUNK // 16
GB = 32
GBP = GB + 16
CB = 128
NEG_INF = float("-inf")


def _seg_body(row_hbm, col_hbm, xvs_hbm, xus_hbm, xev_hbm, xeu_hbm,
              su_hbm, mu_hbm, auxu_hbm, sv_hbm, mv_hbm, auxv_hbm,
              stage_r, stage_c, cpk, ceid,
              gpk0, gpk1, geraw0, geraw1, gcol0, gcol1, gerow0, gerow1,
              vals0, vals1, evals0, evals1,
              acc_s, acc_m, acc_es, acc_em, acc_c,
              smem, sem_a, sem_b, sem_g, sem_g2):
    cid = lax.axis_index("c")
    sid = lax.axis_index("s")
    wid = sid * NC + cid
    lo = wid * RPT

    zeros16 = jnp.zeros((16,), jnp.float32)
    minf16 = jnp.full((16,), NEG_INF, jnp.float32)

    def acc_set(gpk, geraw, vals, evals):
        ones16 = jnp.ones((16,), jnp.float32)

        def acc_one(j, carry):
            d = gpk[pl.ds(j, 16)][0] >> 14
            e = geraw[pl.ds(j, 16)][0]
            for cg in range(8):
                sl = pl.ds(cg * 16, 16)
                v = vals[j, sl]
                plsc.addupdate(acc_s.at[d, sl], v)
                acc_m[d, sl] = jnp.maximum(acc_m[d, sl], v)
            ev = evals[j, pl.ds((e & 7) * 16, 16)]
            d16 = pl.ds(d * 16, 16)
            plsc.addupdate(acc_es.at[d16], ev)
            acc_em[d16] = jnp.maximum(acc_em[d16], ev)
            plsc.addupdate(acc_c.at[d16], ones16)
            return carry
        lax.fori_loop(0, GB, acc_one, 0)

    def run_direction(dst_hbm, src_hbm, table_hbm, etable_hbm,
                      s_hbm, m_hbm, aux_hbm):
        def init_one(r, carry):
            for cg in range(8):
                sl = pl.ds(cg * 16, 16)
                acc_s[r, sl] = zeros16
                acc_m[r, sl] = minf16
            r16 = pl.ds(r * 16, 16)
            acc_es[r16] = zeros16
            acc_em[r16] = minf16
            acc_c[r16] = zeros16
            return carry
        lax.fori_loop(0, ACC_R, init_one, 0)
        smem[0] = 0
        smem[1] = 0
        smem[2] = 0

        def fire_start_set(gpk, geraw, gcol, gerow, vals, evals):
            for p in range(GB // 16):
                sl = pl.ds(p * 16, 16)
                w = cpk[sl]
                e = ceid[sl]
                gpk[sl] = w
                geraw[sl] = e
                gcol[sl] = w & 16383
                gerow[sl] = e >> 3
            pltpu.async_copy(table_hbm.at[gcol], vals, sem_g)
            pltpu.async_copy(etable_hbm.at[gerow], evals, sem_g2)

        def wait_and_acc_pending():
            @pl.when(smem[1] == 1)
            def _():
                pltpu.make_async_copy(
                    table_hbm.at[gcol0], vals0, sem_g).wait()
                pltpu.make_async_copy(
                    etable_hbm.at[gerow0], evals0, sem_g2).wait()

                @pl.when(smem[2] == 1)
                def _():
                    acc_set(gpk0, geraw0, vals0, evals0)

                @pl.when(smem[2] == 0)
                def _():
                    acc_set(gpk1, geraw1, vals1, evals1)
                smem[1] = 0

        def fire():
            wait_and_acc_pending()

            @pl.when(smem[2] == 0)
            def _():
                fire_start_set(gpk0, geraw0, gcol0, gerow0, vals0, evals0)

            @pl.when(smem[2] == 1)
            def _():
                fire_start_set(gpk1, geraw1, gcol1, gerow1, vals1, evals1)
            smem[1] = 1
            smem[2] = 1 - smem[2]

        def stage_start(k, s):
            src_slice = pl.ds(k * CHUNK, CHUNK)

            @pl.when(s == 0)
            def _():
                pltpu.async_copy(dst_hbm.at[src_slice], stage_r.at[pl.ds(0, CHUNK)], sem_a)
                pltpu.async_copy(src_hbm.at[src_slice], stage_c.at[pl.ds(0, CHUNK)], sem_a)

            @pl.when(s == 1)
            def _():
                pltpu.async_copy(dst_hbm.at[src_slice], stage_r.at[pl.ds(CHUNK, CHUNK)], sem_b)
                pltpu.async_copy(src_hbm.at[src_slice], stage_c.at[pl.ds(CHUNK, CHUNK)], sem_b)

        def stage_wait(s):
            @pl.when(s == 0)
            def _():
                pltpu.make_async_copy(
                    dst_hbm.at[pl.ds(0, CHUNK)], stage_r.at[pl.ds(0, CHUNK)], sem_a).wait()
                pltpu.make_async_copy(
                    dst_hbm.at[pl.ds(0, CHUNK)], stage_c.at[pl.ds(0, CHUNK)], sem_a).wait()

            @pl.when(s == 1)
            def _():
                pltpu.make_async_copy(
                    dst_hbm.at[pl.ds(0, CHUNK)], stage_r.at[pl.ds(CHUNK, CHUNK)], sem_b).wait()
                pltpu.make_async_copy(
                    dst_hbm.at[pl.ds(0, CHUNK)], stage_c.at[pl.ds(CHUNK, CHUNK)], sem_b).wait()

        stage_start(0, 0)

        def chunk_body(k, carry):
            base = k * CHUNK
            s = k & 1

            @pl.when(k + 1 < NCHUNKS)
            def _():
                stage_start(k + 1, 1 - s)
            stage_wait(s)

            one16 = jnp.full((16,), 1, jnp.int32)
            zero16 = jnp.zeros((16,), jnp.int32)

            def grp_body(g, off):
                goff = g * 32
                r1 = stage_r[pl.ds(s * CHUNK + goff, 16)]
                c1 = stage_c[pl.ds(s * CHUNK + goff, 16)]
                r2 = stage_r[pl.ds(s * CHUNK + goff + 16, 16)]
                c2 = stage_c[pl.ds(s * CHUNK + goff + 16, 16)]
                m1 = (r1 >= lo) & (r1 < lo + RPT)
                m2 = (r2 >= lo) & (r2 < lo + RPT)
                pfx1 = jnp.cumsum(jnp.where(m1, one16, zero16))
                pfx2 = jnp.cumsum(jnp.where(m2, one16, zero16))
                t1 = pfx1[15]
                idx1 = jnp.maximum(off + pfx1 - 1, 0)
                idx2 = jnp.maximum(off + t1 + pfx2 - 1, 0)
                plsc.store_scatter(cpk, [idx1], c1 | ((r1 - lo) << 14),
                                   mask=m1)
                plsc.store_scatter(cpk, [idx2], c2 | ((r2 - lo) << 14),
                                   mask=m2)
                eidv = lax.iota(jnp.int32, 16) + (base + goff)
                plsc.store_scatter(ceid, [idx1], eidv, mask=m1)
                plsc.store_scatter(ceid, [idx2], eidv + 16, mask=m2)
                offa = off + t1 + pfx2[15]

                @pl.when(offa >= GB)
                def _():
                    fire()
                    for p in range(2):
                        sl_src = pl.ds(GB + p * 16, 16)
                        sl_dst = pl.ds(p * 16, 16)
                        rem_c = cpk[sl_src]
                        rem_e = ceid[sl_src]
                        cpk[sl_dst] = rem_c
                        ceid[sl_dst] = rem_e
                return jnp.where(offa >= GB, offa - GB, offa)

            return lax.fori_loop(0, GRPS // 2, grp_body, carry)

        off = lax.fori_loop(0, NCHUNKS, chunk_body, 0)

        for p in range(GB // 16):
            sl = pl.ds(p * 16, 16)
            lanes = lax.iota(jnp.int32, 16) + (p * 16)
            valid = lanes < off
            cpk[sl] = jnp.where(valid, cpk[sl], TRASH << 14)
            ceid[sl] = jnp.where(valid, ceid[sl], 0)
        fire()
        wait_and_acc_pending()

        pltpu.sync_copy(acc_s.at[pl.ds(0, RPT)], s_hbm.at[pl.ds(lo, RPT)])
        pltpu.sync_copy(acc_m.at[pl.ds(0, RPT)], m_hbm.at[pl.ds(lo, RPT)])
        pltpu.sync_copy(acc_es.at[pl.ds(0, RPT * 16)],
                        aux_hbm.at[pl.ds(lo * 16, RPT * 16)])
        pltpu.sync_copy(acc_em.at[pl.ds(0, RPT * 16)],
                        aux_hbm.at[pl.ds(NPAD * 16 + lo * 16, RPT * 16)])
        pltpu.sync_copy(acc_c.at[pl.ds(0, RPT * 16)],
                        aux_hbm.at[pl.ds(2 * NPAD * 16 + lo * 16, RPT * 16)])

    run_direction(row_hbm, col_hbm, xvs_hbm, xev_hbm,
                  su_hbm, mu_hbm, auxu_hbm)
    run_direction(col_hbm, row_hbm, xus_hbm, xeu_hbm,
                  sv_hbm, mv_hbm, auxv_hbm)


def _sc_segments(row, col, xvs, xus, xe_v2u_f, xe_u2v_f):
    mesh = plsc.VectorSubcoreMesh(core_axis_name="c", subcore_axis_name="s")
    nf = jnp.float32
    out_type = [
        jax.ShapeDtypeStruct((NPAD, 128), nf),
        jax.ShapeDtypeStruct((NPAD, 128), nf),
        jax.ShapeDtypeStruct((3 * NPAD * 16,), nf),
        jax.ShapeDtypeStruct((NPAD, 128), nf),
        jax.ShapeDtypeStruct((NPAD, 128), nf),
        jax.ShapeDtypeStruct((3 * NPAD * 16,), nf),
    ]
    scratch = [
        pltpu.VMEM((2 * CHUNK,), jnp.int32),
        pltpu.VMEM((2 * CHUNK,), jnp.int32),
        pltpu.VMEM((CB,), jnp.int32),
        pltpu.VMEM((CB,), jnp.int32),
        pltpu.VMEM((GBP,), jnp.int32),
        pltpu.VMEM((GBP,), jnp.int32),
        pltpu.VMEM((GBP,), jnp.int32),
        pltpu.VMEM((GBP,), jnp.int32),
        pltpu.VMEM((GB,), jnp.int32),
        pltpu.VMEM((GB,), jnp.int32),
        pltpu.VMEM((GB,), jnp.int32),
        pltpu.VMEM((GB,), jnp.int32),
        pltpu.VMEM((GB, 128), nf),
        pltpu.VMEM((GB, 128), nf),
        pltpu.VMEM((GB, 128), nf),
        pltpu.VMEM((GB, 128), nf),
        pltpu.VMEM((ACC_R, 128), nf),
        pltpu.VMEM((ACC_R, 128), nf),
        pltpu.VMEM((ACC_R * 16,), nf),
        pltpu.VMEM((ACC_R * 16,), nf),
        pltpu.VMEM((ACC_R * 16,), nf),
        pltpu.SMEM((8,), jnp.int32),
        pltpu.SemaphoreType.DMA,
        pltpu.SemaphoreType.DMA,
        pltpu.SemaphoreType.DMA,
        pltpu.SemaphoreType.DMA,
    ]
    run = pl.kernel(_seg_body, out_type=out_type, mesh=mesh,
                    scratch_types=scratch,
                    compiler_params=pltpu.CompilerParams(
                        needs_layout_passes=False))
    return run(row, col, xvs, xus, xe_v2u_f, xe_u2v_f)


_EPT = N_E // NW
_GBATCH = 2000
_NGB = _EPT // _GBATCH


def _puv_body(row_hbm, col_hbm, pu_hbm, pv_hbm, outu_hbm, outv_hbm,
              ridx, cidx, bufu, bufv, sem1, sem2):
    cid = lax.axis_index("c")
    sid = lax.axis_index("s")
    wid = sid * NC + cid

    def batch_body(b, carry):
        base = wid * _EPT + b * _GBATCH
        pltpu.sync_copy(row_hbm.at[pl.ds(base, _GBATCH)], ridx)
        pltpu.sync_copy(col_hbm.at[pl.ds(base, _GBATCH)], cidx)
        d1 = pltpu.async_copy(pu_hbm.at[ridx], bufu, sem1)
        d2 = pltpu.async_copy(pv_hbm.at[cidx], bufv, sem2)
        d1.wait()
        d2.wait()
        pltpu.sync_copy(bufu, outu_hbm.at[pl.ds(base, _GBATCH)])
        pltpu.sync_copy(bufv, outv_hbm.at[pl.ds(base, _GBATCH)])
        return carry

    lax.fori_loop(0, _NGB, batch_body, 0)


def _sc_puv(row, col, pu, pv):
    mesh = plsc.VectorSubcoreMesh(core_axis_name="c", subcore_axis_name="s")
    run = pl.kernel(
        _puv_body,
        out_type=[jax.ShapeDtypeStruct((N_E, 16), jnp.float32),
                  jax.ShapeDtypeStruct((N_E, 16), jnp.float32)],
        mesh=mesh,
        scratch_types=[
            pltpu.VMEM((_GBATCH,), jnp.int32),
            pltpu.VMEM((_GBATCH,), jnp.int32),
            pltpu.VMEM((_GBATCH, 16), jnp.float32),
            pltpu.VMEM((_GBATCH, 16), jnp.float32),
            pltpu.SemaphoreType.DMA,
            pltpu.SemaphoreType.DMA,
        ],
        compiler_params=pltpu.CompilerParams(use_tc_tiling_on_sc=False),
    )
    return run(row, col, pu, pv)


def _node_assemble(x_ref, s_ref, m_ref, es_ref, em_ref, cnt_ref,
                   w1, w2, w3, w4, w5, b_ref, g_ref, be_ref, out_ref):
    cnt = cnt_ref[...][:, :1]
    inv = 1.0 / jnp.maximum(cnt, 1.0)
    nonempty = cnt > 0.0
    mn = s_ref[...] * inv
    mx = jnp.where(nonempty, m_ref[...], 0.0)
    emn = es_ref[...] * inv
    emx = jnp.where(nonempty, em_ref[...], 0.0)
    y = (jnp.dot(x_ref[...], w1[...], preferred_element_type=jnp.float32)
         + jnp.dot(mn, w2[...], preferred_element_type=jnp.float32)
         + jnp.dot(mx, w3[...], preferred_element_type=jnp.float32)
         + jnp.dot(emn, w4[...], preferred_element_type=jnp.float32)
         + jnp.dot(emx, w5[...], preferred_element_type=jnp.float32)
         + b_ref[...])
    mu = jnp.mean(y, axis=0, keepdims=True)
    var = jnp.mean(jnp.square(y - mu), axis=0, keepdims=True)
    out_ref[...] = g_ref[...] * (y - mu) * lax.rsqrt(var + EPS) + be_ref[...]


def _node_out(x, s, m, es, em, cnt, W, b, g, be):
    w1 = W[:, 0:128].T
    w2 = W[:, 128:256].T
    w3 = W[:, 256:384].T
    w4 = W[:, 384:400].T
    w5 = W[:, 400:416].T
    n = x.shape[0]
    return pl.pallas_call(
        _node_assemble,
        out_shape=jax.ShapeDtypeStruct((n, 128), jnp.float32),
    )(x, s, m, es, em, cnt, w1, w2, w3, w4, w5,
      b.reshape(1, 128), g.reshape(1, 128), be.reshape(1, 128))


def _proj_kernel(xu_ref, xv_ref, wu_ref, wv_ref, pu_ref, pv_ref):
    pu_ref[...] = jnp.dot(xu_ref[...], wu_ref[...],
                          preferred_element_type=jnp.float32)
    pv_ref[...] = jnp.dot(xv_ref[...], wv_ref[...],
                          preferred_element_type=jnp.float32)


def _projections(xut, xvt, We):
    return pl.pallas_call(
        _proj_kernel,
        out_shape=[jax.ShapeDtypeStruct((N_U, 16), jnp.float32),
                   jax.ShapeDtypeStruct((N_V, 16), jnp.float32)],
    )(xut, xvt, We[:, 16:144].T, We[:, 144:272].T)


_EBLK = 4000


def _edge_pass1(xe_ref, pu_ref, pv_ref, bd_ref, bias_ref, ye_ref, stats_ref):
    i = pl.program_id(0)
    y = (jnp.dot(xe_ref[...], bd_ref[...], preferred_element_type=jnp.float32)
         + pu_ref[...] + pv_ref[...] + bias_ref[...])
    ye_ref[...] = y

    @pl.when(i == 0)
    def _():
        stats_ref[...] = jnp.zeros_like(stats_ref)

    s = jnp.sum(y, axis=0, keepdims=True)
    ss = jnp.sum(y * y, axis=0, keepdims=True)
    stats_ref[0:1, :] += s
    stats_ref[1:2, :] += ss


def _edge_pass2(ye_ref, scale_ref, shift_ref, out_ref):
    out_ref[...] = ye_ref[...] * scale_ref[...] + shift_ref[...]


def _edge_out(xe_f, pu_f, pv_f, We_e, be, ge, bee):
    bd = jnp.kron(jnp.eye(8, dtype=jnp.float32), We_e.T)
    bias = jnp.tile(be, 8).reshape(1, 128)
    nrows = N_E // 8
    nblk = nrows // _EBLK
    ye, stats = pl.pallas_call(
        _edge_pass1,
        grid=(nblk,),
        in_specs=[
            pl.BlockSpec((_EBLK, 128), lambda i: (i, 0)),
            pl.BlockSpec((_EBLK, 128), lambda i: (i, 0)),
            pl.BlockSpec((_EBLK, 128), lambda i: (i, 0)),
            pl.BlockSpec((128, 128), lambda i: (0, 0)),
            pl.BlockSpec((1, 128), lambda i: (0, 0)),
        ],
        out_specs=[
            pl.BlockSpec((_EBLK, 128), lambda i: (i, 0)),
            pl.BlockSpec((8, 128), lambda i: (0, 0)),
        ],
        out_shape=[
            jax.ShapeDtypeStruct((nrows, 128), jnp.float32),
            jax.ShapeDtypeStruct((8, 128), jnp.float32),
        ],
    )(xe_f, pu_f, pv_f, bd, bias)
    s16 = jnp.sum(stats[0].reshape(8, 16), axis=0)
    ss16 = jnp.sum(stats[1].reshape(8, 16), axis=0)
    mu = s16 / N_E
    var = ss16 / N_E - mu * mu
    scale = ge * lax.rsqrt(var + EPS)
    shift = bee - mu * scale
    out = pl.pallas_call(
        _edge_pass2,
        grid=(nblk,),
        in_specs=[
            pl.BlockSpec((_EBLK, 128), lambda i: (i, 0)),
            pl.BlockSpec((1, 128), lambda i: (0, 0)),
            pl.BlockSpec((1, 128), lambda i: (0, 0)),
        ],
        out_specs=pl.BlockSpec((_EBLK, 128), lambda i: (i, 0)),
        out_shape=jax.ShapeDtypeStruct((nrows, 128), jnp.float32),
    )(ye, jnp.tile(scale, 8).reshape(1, 128), jnp.tile(shift, 8).reshape(1, 128))
    return out.reshape(N_E, 16)


def kernel(xus, xut, xvs, xvt, edge_index, xe_e, xe_v2u, xe_u2v,
           Wu, bu, gu, beu, Wv, bv, gv, bev, We, be, ge, bee):
    row = jnp.asarray(edge_index[0], jnp.int32)
    col = jnp.asarray(edge_index[1], jnp.int32)

    (s_u, m_u, aux_u,
     s_v, m_v, aux_v) = _sc_segments(row, col, xvs, xus,
                                     xe_v2u.reshape(N_E // 8, 128),
                                     xe_u2v.reshape(N_E // 8, 128))

    def unaux(aux):
        es = aux[0:NPAD * 16].reshape(NPAD, 16)
        em = aux[NPAD * 16:2 * NPAD * 16].reshape(NPAD, 16)
        c = aux[2 * NPAD * 16:].reshape(NPAD, 16)
        return es[:N_U], em[:N_U], c[:N_U]

    es_u, em_u, cnt_u = unaux(aux_u)
    es_v, em_v, cnt_v = unaux(aux_v)

    out_u = _node_out(xut, s_u[:N_U], m_u[:N_U], es_u, em_u, cnt_u,
                      Wu, bu, gu, beu)
    out_v = _node_out(xvt, s_v[:N_V], m_v[:N_V], es_v, em_v, cnt_v,
                      Wv, bv, gv, bev)

    pu, pv = _projections(xut, xvt, We)
    pu_e, pv_e = _sc_puv(row, col, pu, pv)
    out_e = _edge_out(xe_e.reshape(N_E // 8, 128),
                      pu_e.reshape(N_E // 8, 128),
                      pv_e.reshape(N_E // 8, 128),
                      We[:, 0:16], be, ge, bee)
    return (out_u, out_v, out_e)


# speedup vs baseline: 1.0079x; 1.0079x over previous
import functools

import jax
import jax.numpy as jnp
from jax import lax
from jax.experimental import pallas as pl
from jax.experimental.pallas import tpu as pltpu
from jax.experimental.pallas import tpu_sc as plsc

N_U = 10000
N_V = 10000
N_E = 320000
D_N = 128
D_E = 16
EPS = 1e-5

NC = 2
NS = 16
NW = NC * NS
RPT = 320
NPAD = NW * RPT
TRASH = RPT
ACC_R = RPT + 8
CHUNK = 1280
NCHUNKS = N_E // CHUNK
GRPS = CHUNK // 16
GB = 32
GBP = GB + 16
CB = 128
NEG_INF = float("-inf")


def _seg_body(row_hbm, col_hbm, xvs_hbm, xus_hbm, xev_hbm, xeu_hbm,
              su_hbm, mu_hbm, auxu_hbm, sv_hbm, mv_hbm, auxv_hbm,
              stage_r, stage_c, cpk, ceid,
              gpk0, gpk1, geraw0, geraw1, gcol0, gcol1, gerow0, gerow1,
              vals0, vals1, evals0, evals1,
              acc_s, acc_m, acc_es, acc_em, acc_c,
              smem, sem_a, sem_b, sem_g, sem_g2):
    cid = lax.axis_index("c")
    sid = lax.axis_index("s")
    wid = sid * NC + cid
    lo = wid * RPT

    zeros16 = jnp.zeros((16,), jnp.float32)
    minf16 = jnp.full((16,), NEG_INF, jnp.float32)

    def acc_set(gpk, geraw, vals, evals):
        ones16 = jnp.ones((16,), jnp.float32)

        def acc_one(j, carry):
            d = gpk[pl.ds(j, 16)][0] >> 14
            e = geraw[pl.ds(j, 16)][0]
            for cg in range(8):
                sl = pl.ds(cg * 16, 16)
                v = vals[j, sl]
                plsc.addupdate(acc_s.at[d, sl], v)
                acc_m[d, sl] = jnp.maximum(acc_m[d, sl], v)
            ev = evals[j, pl.ds((e & 7) * 16, 16)]
            d16 = pl.ds(d * 16, 16)
            plsc.addupdate(acc_es.at[d16], ev)
            acc_em[d16] = jnp.maximum(acc_em[d16], ev)
            plsc.addupdate(acc_c.at[d16], ones16)
            return carry
        lax.fori_loop(0, GB, acc_one, 0)

    def run_direction(dst_hbm, src_hbm, table_hbm, etable_hbm,
                      s_hbm, m_hbm, aux_hbm):
        def init_one(r, carry):
            for cg in range(8):
                sl = pl.ds(cg * 16, 16)
                acc_s[r, sl] = zeros16
                acc_m[r, sl] = minf16
            r16 = pl.ds(r * 16, 16)
            acc_es[r16] = zeros16
            acc_em[r16] = minf16
            acc_c[r16] = zeros16
            return carry
        lax.fori_loop(0, ACC_R, init_one, 0)
        smem[0] = 0
        smem[1] = 0
        smem[2] = 0

        def fire_start_set(gpk, geraw, gcol, gerow, vals, evals):
            for p in range(GB // 16):
                sl = pl.ds(p * 16, 16)
                w = cpk[sl]
                e = ceid[sl]
                gpk[sl] = w
                geraw[sl] = e
                gcol[sl] = w & 16383
                gerow[sl] = e >> 3
            pltpu.async_copy(table_hbm.at[gcol], vals, sem_g)
            pltpu.async_copy(etable_hbm.at[gerow], evals, sem_g2)

        def wait_and_acc_pending():
            @pl.when(smem[1] == 1)
            def _():
                pltpu.make_async_copy(
                    table_hbm.at[gcol0], vals0, sem_g).wait()
                pltpu.make_async_copy(
                    etable_hbm.at[gerow0], evals0, sem_g2).wait()

                @pl.when(smem[2] == 1)
                def _():
                    acc_set(gpk0, geraw0, vals0, evals0)

                @pl.when(smem[2] == 0)
                def _():
                    acc_set(gpk1, geraw1, vals1, evals1)
                smem[1] = 0

        def fire():
            wait_and_acc_pending()

            @pl.when(smem[2] == 0)
            def _():
                fire_start_set(gpk0, geraw0, gcol0, gerow0, vals0, evals0)

            @pl.when(smem[2] == 1)
            def _():
                fire_start_set(gpk1, geraw1, gcol1, gerow1, vals1, evals1)
            smem[1] = 1
            smem[2] = 1 - smem[2]

        def stage_start(k, s):
            src_slice = pl.ds(k * CHUNK, CHUNK)

            @pl.when(s == 0)
            def _():
                pltpu.async_copy(dst_hbm.at[src_slice], stage_r.at[pl.ds(0, CHUNK)], sem_a)
                pltpu.async_copy(src_hbm.at[src_slice], stage_c.at[pl.ds(0, CHUNK)], sem_a)

            @pl.when(s == 1)
            def _():
                pltpu.async_copy(dst_hbm.at[src_slice], stage_r.at[pl.ds(CHUNK, CHUNK)], sem_b)
                pltpu.async_copy(src_hbm.at[src_slice], stage_c.at[pl.ds(CHUNK, CHUNK)], sem_b)

        def stage_wait(s):
            @pl.when(s == 0)
            def _():
                pltpu.make_async_copy(
                    dst_hbm.at[pl.ds(0, CHUNK)], stage_r.at[pl.ds(0, CHUNK)], sem_a).wait()
                pltpu.make_async_copy(
                    dst_hbm.at[pl.ds(0, CHUNK)], stage_c.at[pl.ds(0, CHUNK)], sem_a).wait()

            @pl.when(s == 1)
            def _():
                pltpu.make_async_copy(
                    dst_hbm.at[pl.ds(0, CHUNK)], stage_r.at[pl.ds(CHUNK, CHUNK)], sem_b).wait()
                pltpu.make_async_copy(
                    dst_hbm.at[pl.ds(0, CHUNK)], stage_c.at[pl.ds(CHUNK, CHUNK)], sem_b).wait()

        stage_start(0, 0)

        def chunk_body(k, carry):
            base = k * CHUNK
            s = k & 1

            @pl.when(k + 1 < NCHUNKS)
            def _():
                stage_start(k + 1, 1 - s)
            stage_wait(s)

            one16 = jnp.full((16,), 1, jnp.int32)
            zero16 = jnp.zeros((16,), jnp.int32)

            def grp_body(g, off):
                goff = g * 32
                r1 = stage_r[pl.ds(s * CHUNK + goff, 16)]
                c1 = stage_c[pl.ds(s * CHUNK + goff, 16)]
                r2 = stage_r[pl.ds(s * CHUNK + goff + 16, 16)]
                c2 = stage_c[pl.ds(s * CHUNK + goff + 16, 16)]
                m1 = (r1 >= lo) & (r1 < lo + RPT)
                m2 = (r2 >= lo) & (r2 < lo + RPT)
                pfx1 = jnp.cumsum(jnp.where(m1, one16, zero16))
                pfx2 = jnp.cumsum(jnp.where(m2, one16, zero16))
                t1 = pfx1[15]
                idx1 = jnp.maximum(off + pfx1 - 1, 0)
                idx2 = jnp.maximum(off + t1 + pfx2 - 1, 0)
                plsc.store_scatter(cpk, [idx1], c1 | ((r1 - lo) << 14),
                                   mask=m1)
                plsc.store_scatter(cpk, [idx2], c2 | ((r2 - lo) << 14),
                                   mask=m2)
                eidv = lax.iota(jnp.int32, 16) + (base + goff)
                plsc.store_scatter(ceid, [idx1], eidv, mask=m1)
                plsc.store_scatter(ceid, [idx2], eidv + 16, mask=m2)
                offa = off + t1 + pfx2[15]

                @pl.when(offa >= GB)
                def _():
                    fire()
                    for p in range(2):
                        sl_src = pl.ds(GB + p * 16, 16)
                        sl_dst = pl.ds(p * 16, 16)
                        rem_c = cpk[sl_src]
                        rem_e = ceid[sl_src]
                        cpk[sl_dst] = rem_c
                        ceid[sl_dst] = rem_e
                return jnp.where(offa >= GB, offa - GB, offa)

            return lax.fori_loop(0, GRPS // 2, grp_body, carry)

        off = lax.fori_loop(0, NCHUNKS, chunk_body, 0)

        for p in range(GB // 16):
            sl = pl.ds(p * 16, 16)
            lanes = lax.iota(jnp.int32, 16) + (p * 16)
            valid = lanes < off
            cpk[sl] = jnp.where(valid, cpk[sl], TRASH << 14)
            ceid[sl] = jnp.where(valid, ceid[sl], 0)
        fire()
        wait_and_acc_pending()

        pltpu.sync_copy(acc_s.at[pl.ds(0, RPT)], s_hbm.at[pl.ds(lo, RPT)])
        pltpu.sync_copy(acc_m.at[pl.ds(0, RPT)], m_hbm.at[pl.ds(lo, RPT)])
        pltpu.sync_copy(acc_es.at[pl.ds(0, RPT * 16)],
                        aux_hbm.at[pl.ds(lo * 16, RPT * 16)])
        pltpu.sync_copy(acc_em.at[pl.ds(0, RPT * 16)],
                        aux_hbm.at[pl.ds(NPAD * 16 + lo * 16, RPT * 16)])
        pltpu.sync_copy(acc_c.at[pl.ds(0, RPT * 16)],
                        aux_hbm.at[pl.ds(2 * NPAD * 16 + lo * 16, RPT * 16)])

    run_direction(row_hbm, col_hbm, xvs_hbm, xev_hbm,
                  su_hbm, mu_hbm, auxu_hbm)
    run_direction(col_hbm, row_hbm, xus_hbm, xeu_hbm,
                  sv_hbm, mv_hbm, auxv_hbm)


def _sc_segments(row, col, xvs, xus, xe_v2u_f, xe_u2v_f):
    mesh = plsc.VectorSubcoreMesh(core_axis_name="c", subcore_axis_name="s")
    nf = jnp.float32
    out_type = [
        jax.ShapeDtypeStruct((NPAD, 128), nf),
        jax.ShapeDtypeStruct((NPAD, 128), nf),
        jax.ShapeDtypeStruct((3 * NPAD * 16,), nf),
        jax.ShapeDtypeStruct((NPAD, 128), nf),
        jax.ShapeDtypeStruct((NPAD, 128), nf),
        jax.ShapeDtypeStruct((3 * NPAD * 16,), nf),
    ]
    scratch = [
        pltpu.VMEM((2 * CHUNK,), jnp.int32),
        pltpu.VMEM((2 * CHUNK,), jnp.int32),
        pltpu.VMEM((CB,), jnp.int32),
        pltpu.VMEM((CB,), jnp.int32),
        pltpu.VMEM((GBP,), jnp.int32),
        pltpu.VMEM((GBP,), jnp.int32),
        pltpu.VMEM((GBP,), jnp.int32),
        pltpu.VMEM((GBP,), jnp.int32),
        pltpu.VMEM((GB,), jnp.int32),
        pltpu.VMEM((GB,), jnp.int32),
        pltpu.VMEM((GB,), jnp.int32),
        pltpu.VMEM((GB,), jnp.int32),
        pltpu.VMEM((GB, 128), nf),
        pltpu.VMEM((GB, 128), nf),
        pltpu.VMEM((GB, 128), nf),
        pltpu.VMEM((GB, 128), nf),
        pltpu.VMEM((ACC_R, 128), nf),
        pltpu.VMEM((ACC_R, 128), nf),
        pltpu.VMEM((ACC_R * 16,), nf),
        pltpu.VMEM((ACC_R * 16,), nf),
        pltpu.VMEM((ACC_R * 16,), nf),
        pltpu.SMEM((8,), jnp.int32),
        pltpu.SemaphoreType.DMA,
        pltpu.SemaphoreType.DMA,
        pltpu.SemaphoreType.DMA,
        pltpu.SemaphoreType.DMA,
    ]
    run = pl.kernel(_seg_body, out_type=out_type, mesh=mesh,
                    scratch_types=scratch,
                    compiler_params=pltpu.CompilerParams(
                        needs_layout_passes=False))
    return run(row, col, xvs, xus, xe_v2u_f, xe_u2v_f)


_EPT = N_E // NW
_GBATCH = 2000
_NGB = _EPT // _GBATCH


def _puv_body(row_hbm, col_hbm, pu_hbm, pv_hbm, outu_hbm, outv_hbm,
              ridx, cidx, bufu, bufv, sem1, sem2):
    cid = lax.axis_index("c")
    sid = lax.axis_index("s")
    wid = sid * NC + cid

    def batch_body(b, carry):
        base = wid * _EPT + b * _GBATCH
        pltpu.sync_copy(row_hbm.at[pl.ds(base, _GBATCH)], ridx)
        pltpu.sync_copy(col_hbm.at[pl.ds(base, _GBATCH)], cidx)
        d1 = pltpu.async_copy(pu_hbm.at[ridx], bufu, sem1)
        d2 = pltpu.async_copy(pv_hbm.at[cidx], bufv, sem2)
        d1.wait()
        d2.wait()
        pltpu.sync_copy(bufu, outu_hbm.at[pl.ds(base, _GBATCH)])
        pltpu.sync_copy(bufv, outv_hbm.at[pl.ds(base, _GBATCH)])
        return carry

    lax.fori_loop(0, _NGB, batch_body, 0)


def _sc_puv(row, col, pu, pv):
    mesh = plsc.VectorSubcoreMesh(core_axis_name="c", subcore_axis_name="s")
    run = pl.kernel(
        _puv_body,
        out_type=[jax.ShapeDtypeStruct((N_E, 16), jnp.float32),
                  jax.ShapeDtypeStruct((N_E, 16), jnp.float32)],
        mesh=mesh,
        scratch_types=[
            pltpu.VMEM((_GBATCH,), jnp.int32),
            pltpu.VMEM((_GBATCH,), jnp.int32),
            pltpu.VMEM((_GBATCH, 16), jnp.float32),
            pltpu.VMEM((_GBATCH, 16), jnp.float32),
            pltpu.SemaphoreType.DMA,
            pltpu.SemaphoreType.DMA,
        ],
        compiler_params=pltpu.CompilerParams(use_tc_tiling_on_sc=False),
    )
    return run(row, col, pu, pv)


def _node_assemble(x_ref, s_ref, m_ref, es_ref, em_ref, cnt_ref,
                   w1, w2, w3, w4, w5, b_ref, g_ref, be_ref, out_ref):
    n = out_ref.shape[0]
    cnt = cnt_ref[...][:, :1]
    inv = 1.0 / jnp.maximum(cnt, 1.0)
    nonempty = cnt > 0.0
    mn = s_ref[...] * inv
    mx = jnp.where(nonempty, m_ref[...], 0.0)
    emn = es_ref[...] * inv
    emx = jnp.where(nonempty, em_ref[...], 0.0)
    y = (jnp.dot(x_ref[...], w1[...], preferred_element_type=jnp.float32)
         + jnp.dot(mn[0:n], w2[...], preferred_element_type=jnp.float32)
         + jnp.dot(mx[0:n], w3[...], preferred_element_type=jnp.float32)
         + jnp.dot(emn[0:n], w4[...], preferred_element_type=jnp.float32)
         + jnp.dot(emx[0:n], w5[...], preferred_element_type=jnp.float32)
         + b_ref[...])
    mu = jnp.mean(y, axis=0, keepdims=True)
    var = jnp.mean(jnp.square(y - mu), axis=0, keepdims=True)
    out_ref[...] = g_ref[...] * (y - mu) * lax.rsqrt(var + EPS) + be_ref[...]


def _node_out(x, s, m, es, em, cnt, W, b, g, be):
    w1 = W[:, 0:128].T
    w2 = W[:, 128:256].T
    w3 = W[:, 256:384].T
    w4 = W[:, 384:400].T
    w5 = W[:, 400:416].T
    n = x.shape[0]
    return pl.pallas_call(
        _node_assemble,
        out_shape=jax.ShapeDtypeStruct((n, 128), jnp.float32),
    )(x, s, m, es, em, cnt, w1, w2, w3, w4, w5,
      b.reshape(1, 128), g.reshape(1, 128), be.reshape(1, 128))


def _proj_kernel(xu_ref, xv_ref, wu_ref, wv_ref, pu_ref, pv_ref):
    pu_ref[...] = jnp.dot(xu_ref[...], wu_ref[...],
                          preferred_element_type=jnp.float32)
    pv_ref[...] = jnp.dot(xv_ref[...], wv_ref[...],
                          preferred_element_type=jnp.float32)


def _projections(xut, xvt, We):
    return pl.pallas_call(
        _proj_kernel,
        out_shape=[jax.ShapeDtypeStruct((N_U, 16), jnp.float32),
                   jax.ShapeDtypeStruct((N_V, 16), jnp.float32)],
    )(xut, xvt, We[:, 16:144].T, We[:, 144:272].T)


_EBLK = 4000


def _edge_pass1(xe_ref, pu_ref, pv_ref, bd_ref, bias_ref, ye_ref, stats_ref):
    i = pl.program_id(0)
    y = (jnp.dot(xe_ref[...], bd_ref[...], preferred_element_type=jnp.float32)
         + pu_ref[...] + pv_ref[...] + bias_ref[...])
    ye_ref[...] = y

    @pl.when(i == 0)
    def _():
        stats_ref[...] = jnp.zeros_like(stats_ref)

    s = jnp.sum(y, axis=0, keepdims=True)
    ss = jnp.sum(y * y, axis=0, keepdims=True)
    stats_ref[0:1, :] += s
    stats_ref[1:2, :] += ss


def _edge_pass2(ye_ref, scale_ref, shift_ref, out_ref):
    out_ref[...] = ye_ref[...] * scale_ref[...] + shift_ref[...]


def _edge_out(xe_f, pu_f, pv_f, We_e, be, ge, bee):
    bd = jnp.kron(jnp.eye(8, dtype=jnp.float32), We_e.T)
    bias = jnp.tile(be, 8).reshape(1, 128)
    nrows = N_E // 8
    nblk = nrows // _EBLK
    ye, stats = pl.pallas_call(
        _edge_pass1,
        grid=(nblk,),
        in_specs=[
            pl.BlockSpec((_EBLK, 128), lambda i: (i, 0)),
            pl.BlockSpec((_EBLK, 128), lambda i: (i, 0)),
            pl.BlockSpec((_EBLK, 128), lambda i: (i, 0)),
            pl.BlockSpec((128, 128), lambda i: (0, 0)),
            pl.BlockSpec((1, 128), lambda i: (0, 0)),
        ],
        out_specs=[
            pl.BlockSpec((_EBLK, 128), lambda i: (i, 0)),
            pl.BlockSpec((8, 128), lambda i: (0, 0)),
        ],
        out_shape=[
            jax.ShapeDtypeStruct((nrows, 128), jnp.float32),
            jax.ShapeDtypeStruct((8, 128), jnp.float32),
        ],
    )(xe_f, pu_f, pv_f, bd, bias)
    s16 = jnp.sum(stats[0].reshape(8, 16), axis=0)
    ss16 = jnp.sum(stats[1].reshape(8, 16), axis=0)
    mu = s16 / N_E
    var = ss16 / N_E - mu * mu
    scale = ge * lax.rsqrt(var + EPS)
    shift = bee - mu * scale
    out = pl.pallas_call(
        _edge_pass2,
        grid=(nblk,),
        in_specs=[
            pl.BlockSpec((_EBLK, 128), lambda i: (i, 0)),
            pl.BlockSpec((1, 128), lambda i: (0, 0)),
            pl.BlockSpec((1, 128), lambda i: (0, 0)),
        ],
        out_specs=pl.BlockSpec((_EBLK, 128), lambda i: (i, 0)),
        out_shape=jax.ShapeDtypeStruct((nrows, 128), jnp.float32),
    )(ye, jnp.tile(scale, 8).reshape(1, 128), jnp.tile(shift, 8).reshape(1, 128))
    return out.reshape(N_E, 16)


def kernel(xus, xut, xvs, xvt, edge_index, xe_e, xe_v2u, xe_u2v,
           Wu, bu, gu, beu, Wv, bv, gv, bev, We, be, ge, bee):
    row = jnp.asarray(edge_index[0], jnp.int32)
    col = jnp.asarray(edge_index[1], jnp.int32)

    (s_u, m_u, aux_u,
     s_v, m_v, aux_v) = _sc_segments(row, col, xvs, xus,
                                     xe_v2u.reshape(N_E // 8, 128),
                                     xe_u2v.reshape(N_E // 8, 128))

    def unaux(aux):
        es = aux[0:NPAD * 16].reshape(NPAD, 16)
        em = aux[NPAD * 16:2 * NPAD * 16].reshape(NPAD, 16)
        c = aux[2 * NPAD * 16:].reshape(NPAD, 16)
        return es, em, c

    es_u, em_u, cnt_u = unaux(aux_u)
    es_v, em_v, cnt_v = unaux(aux_v)

    out_u = _node_out(xut, s_u, m_u, es_u, em_u, cnt_u, Wu, bu, gu, beu)
    out_v = _node_out(xvt, s_v, m_v, es_v, em_v, cnt_v, Wv, bv, gv, bev)

    pu, pv = _projections(xut, xvt, We)
    pu_e, pv_e = _sc_puv(row, col, pu, pv)
    out_e = _edge_out(xe_e.reshape(N_E // 8, 128),
                      pu_e.reshape(N_E // 8, 128),
                      pv_e.reshape(N_E // 8, 128),
                      We[:, 0:16], be, ge, bee)
    return (out_u, out_v, out_e)


# speedup vs baseline: 1.0092x; 1.0013x over previous
import functools

import jax
import jax.numpy as jnp
from jax import lax
from jax.experimental import pallas as pl
from jax.experimental.pallas import tpu as pltpu
from jax.experimental.pallas import tpu_sc as plsc

N_U = 10000
N_V = 10000
N_E = 320000
D_N = 128
D_E = 16
EPS = 1e-5

NC = 2
NS = 16
NW = NC * NS
RPT = 320
NPAD = NW * RPT
TRASH = RPT
ACC_R = RPT + 8
CHUNK = 2560
NCHUNKS = N_E // CHUNK
GRPS = CHUNK // 16
GB = 32
GBP = GB + 16
CB = 128
NEG_INF = float("-inf")


def _seg_body(row_hbm, col_hbm, xvs_hbm, xus_hbm, xev_hbm, xeu_hbm,
              su_hbm, mu_hbm, auxu_hbm, sv_hbm, mv_hbm, auxv_hbm,
              stage_r, stage_c, cpk, ceid,
              gpk0, gpk1, geraw0, geraw1, gcol0, gcol1, gerow0, gerow1,
              vals0, vals1, evals0, evals1,
              acc_s, acc_m, acc_es, acc_em, acc_c,
              smem, sem_a, sem_b, sem_g, sem_g2):
    cid = lax.axis_index("c")
    sid = lax.axis_index("s")
    wid = sid * NC + cid
    lo = wid * RPT

    zeros16 = jnp.zeros((16,), jnp.float32)
    minf16 = jnp.full((16,), NEG_INF, jnp.float32)

    def acc_set(gpk, geraw, vals, evals):
        ones16 = jnp.ones((16,), jnp.float32)

        def acc_one(j, carry):
            d = gpk[pl.ds(j, 16)][0] >> 14
            e = geraw[pl.ds(j, 16)][0]
            for cg in range(8):
                sl = pl.ds(cg * 16, 16)
                v = vals[j, sl]
                plsc.addupdate(acc_s.at[d, sl], v)
                acc_m[d, sl] = jnp.maximum(acc_m[d, sl], v)
            ev = evals[j, pl.ds((e & 7) * 16, 16)]
            d16 = pl.ds(d * 16, 16)
            plsc.addupdate(acc_es.at[d16], ev)
            acc_em[d16] = jnp.maximum(acc_em[d16], ev)
            plsc.addupdate(acc_c.at[d16], ones16)
            return carry
        lax.fori_loop(0, GB, acc_one, 0)

    def run_direction(dst_hbm, src_hbm, table_hbm, etable_hbm,
                      s_hbm, m_hbm, aux_hbm):
        def init_one(r, carry):
            for cg in range(8):
                sl = pl.ds(cg * 16, 16)
                acc_s[r, sl] = zeros16
                acc_m[r, sl] = minf16
            r16 = pl.ds(r * 16, 16)
            acc_es[r16] = zeros16
            acc_em[r16] = minf16
            acc_c[r16] = zeros16
            return carry
        lax.fori_loop(0, ACC_R, init_one, 0)
        smem[0] = 0
        smem[1] = 0
        smem[2] = 0

        def fire_start_set(gpk, geraw, gcol, gerow, vals, evals):
            for p in range(GB // 16):
                sl = pl.ds(p * 16, 16)
                w = cpk[sl]
                e = ceid[sl]
                gpk[sl] = w
                geraw[sl] = e
                gcol[sl] = w & 16383
                gerow[sl] = e >> 3
            pltpu.async_copy(table_hbm.at[gcol], vals, sem_g)
            pltpu.async_copy(etable_hbm.at[gerow], evals, sem_g2)

        def wait_and_acc_pending():
            @pl.when(smem[1] == 1)
            def _():
                pltpu.make_async_copy(
                    table_hbm.at[gcol0], vals0, sem_g).wait()
                pltpu.make_async_copy(
                    etable_hbm.at[gerow0], evals0, sem_g2).wait()

                @pl.when(smem[2] == 1)
                def _():
                    acc_set(gpk0, geraw0, vals0, evals0)

                @pl.when(smem[2] == 0)
                def _():
                    acc_set(gpk1, geraw1, vals1, evals1)
                smem[1] = 0

        def fire():
            wait_and_acc_pending()

            @pl.when(smem[2] == 0)
            def _():
                fire_start_set(gpk0, geraw0, gcol0, gerow0, vals0, evals0)

            @pl.when(smem[2] == 1)
            def _():
                fire_start_set(gpk1, geraw1, gcol1, gerow1, vals1, evals1)
            smem[1] = 1
            smem[2] = 1 - smem[2]

        def stage_start(k, s):
            src_slice = pl.ds(k * CHUNK, CHUNK)

            @pl.when(s == 0)
            def _():
                pltpu.async_copy(dst_hbm.at[src_slice], stage_r.at[pl.ds(0, CHUNK)], sem_a)
                pltpu.async_copy(src_hbm.at[src_slice], stage_c.at[pl.ds(0, CHUNK)], sem_a)

            @pl.when(s == 1)
            def _():
                pltpu.async_copy(dst_hbm.at[src_slice], stage_r.at[pl.ds(CHUNK, CHUNK)], sem_b)
                pltpu.async_copy(src_hbm.at[src_slice], stage_c.at[pl.ds(CHUNK, CHUNK)], sem_b)

        def stage_wait(s):
            @pl.when(s == 0)
            def _():
                pltpu.make_async_copy(
                    dst_hbm.at[pl.ds(0, CHUNK)], stage_r.at[pl.ds(0, CHUNK)], sem_a).wait()
                pltpu.make_async_copy(
                    dst_hbm.at[pl.ds(0, CHUNK)], stage_c.at[pl.ds(0, CHUNK)], sem_a).wait()

            @pl.when(s == 1)
            def _():
                pltpu.make_async_copy(
                    dst_hbm.at[pl.ds(0, CHUNK)], stage_r.at[pl.ds(CHUNK, CHUNK)], sem_b).wait()
                pltpu.make_async_copy(
                    dst_hbm.at[pl.ds(0, CHUNK)], stage_c.at[pl.ds(CHUNK, CHUNK)], sem_b).wait()

        stage_start(0, 0)

        def chunk_body(k, carry):
            base = k * CHUNK
            s = k & 1

            @pl.when(k + 1 < NCHUNKS)
            def _():
                stage_start(k + 1, 1 - s)
            stage_wait(s)

            one16 = jnp.full((16,), 1, jnp.int32)
            zero16 = jnp.zeros((16,), jnp.int32)

            def grp_body(g, off):
                goff = g * 32
                r1 = stage_r[pl.ds(s * CHUNK + goff, 16)]
                c1 = stage_c[pl.ds(s * CHUNK + goff, 16)]
                r2 = stage_r[pl.ds(s * CHUNK + goff + 16, 16)]
                c2 = stage_c[pl.ds(s * CHUNK + goff + 16, 16)]
                m1 = (r1 >= lo) & (r1 < lo + RPT)
                m2 = (r2 >= lo) & (r2 < lo + RPT)
                pfx1 = jnp.cumsum(jnp.where(m1, one16, zero16))
                pfx2 = jnp.cumsum(jnp.where(m2, one16, zero16))
                t1 = pfx1[15]
                idx1 = jnp.maximum(off + pfx1 - 1, 0)
                idx2 = jnp.maximum(off + t1 + pfx2 - 1, 0)
                plsc.store_scatter(cpk, [idx1], c1 | ((r1 - lo) << 14),
                                   mask=m1)
                plsc.store_scatter(cpk, [idx2], c2 | ((r2 - lo) << 14),
                                   mask=m2)
                eidv = lax.iota(jnp.int32, 16) + (base + goff)
                plsc.store_scatter(ceid, [idx1], eidv, mask=m1)
                plsc.store_scatter(ceid, [idx2], eidv + 16, mask=m2)
                offa = off + t1 + pfx2[15]

                @pl.when(offa >= GB)
                def _():
                    fire()
                    for p in range(2):
                        sl_src = pl.ds(GB + p * 16, 16)
                        sl_dst = pl.ds(p * 16, 16)
                        rem_c = cpk[sl_src]
                        rem_e = ceid[sl_src]
                        cpk[sl_dst] = rem_c
                        ceid[sl_dst] = rem_e
                return jnp.where(offa >= GB, offa - GB, offa)

            return lax.fori_loop(0, GRPS // 2, grp_body, carry)

        off = lax.fori_loop(0, NCHUNKS, chunk_body, 0)

        for p in range(GB // 16):
            sl = pl.ds(p * 16, 16)
            lanes = lax.iota(jnp.int32, 16) + (p * 16)
            valid = lanes < off
            cpk[sl] = jnp.where(valid, cpk[sl], TRASH << 14)
            ceid[sl] = jnp.where(valid, ceid[sl], 0)
        fire()
        wait_and_acc_pending()

        pltpu.sync_copy(acc_s.at[pl.ds(0, RPT)], s_hbm.at[pl.ds(lo, RPT)])
        pltpu.sync_copy(acc_m.at[pl.ds(0, RPT)], m_hbm.at[pl.ds(lo, RPT)])
        pltpu.sync_copy(acc_es.at[pl.ds(0, RPT * 16)],
                        aux_hbm.at[pl.ds(lo * 16, RPT * 16)])
        pltpu.sync_copy(acc_em.at[pl.ds(0, RPT * 16)],
                        aux_hbm.at[pl.ds(NPAD * 16 + lo * 16, RPT * 16)])
        pltpu.sync_copy(acc_c.at[pl.ds(0, RPT * 16)],
                        aux_hbm.at[pl.ds(2 * NPAD * 16 + lo * 16, RPT * 16)])

    run_direction(row_hbm, col_hbm, xvs_hbm, xev_hbm,
                  su_hbm, mu_hbm, auxu_hbm)
    run_direction(col_hbm, row_hbm, xus_hbm, xeu_hbm,
                  sv_hbm, mv_hbm, auxv_hbm)


def _sc_segments(row, col, xvs, xus, xe_v2u_f, xe_u2v_f):
    mesh = plsc.VectorSubcoreMesh(core_axis_name="c", subcore_axis_name="s")
    nf = jnp.float32
    out_type = [
        jax.ShapeDtypeStruct((NPAD, 128), nf),
        jax.ShapeDtypeStruct((NPAD, 128), nf),
        jax.ShapeDtypeStruct((3 * NPAD * 16,), nf),
        jax.ShapeDtypeStruct((NPAD, 128), nf),
        jax.ShapeDtypeStruct((NPAD, 128), nf),
        jax.ShapeDtypeStruct((3 * NPAD * 16,), nf),
    ]
    scratch = [
        pltpu.VMEM((2 * CHUNK,), jnp.int32),
        pltpu.VMEM((2 * CHUNK,), jnp.int32),
        pltpu.VMEM((CB,), jnp.int32),
        pltpu.VMEM((CB,), jnp.int32),
        pltpu.VMEM((GBP,), jnp.int32),
        pltpu.VMEM((GBP,), jnp.int32),
        pltpu.VMEM((GBP,), jnp.int32),
        pltpu.VMEM((GBP,), jnp.int32),
        pltpu.VMEM((GB,), jnp.int32),
        pltpu.VMEM((GB,), jnp.int32),
        pltpu.VMEM((GB,), jnp.int32),
        pltpu.VMEM((GB,), jnp.int32),
        pltpu.VMEM((GB, 128), nf),
        pltpu.VMEM((GB, 128), nf),
        pltpu.VMEM((GB, 128), nf),
        pltpu.VMEM((GB, 128), nf),
        pltpu.VMEM((ACC_R, 128), nf),
        pltpu.VMEM((ACC_R, 128), nf),
        pltpu.VMEM((ACC_R * 16,), nf),
        pltpu.VMEM((ACC_R * 16,), nf),
        pltpu.VMEM((ACC_R * 16,), nf),
        pltpu.SMEM((8,), jnp.int32),
        pltpu.SemaphoreType.DMA,
        pltpu.SemaphoreType.DMA,
        pltpu.SemaphoreType.DMA,
        pltpu.SemaphoreType.DMA,
    ]
    run = pl.kernel(_seg_body, out_type=out_type, mesh=mesh,
                    scratch_types=scratch,
                    compiler_params=pltpu.CompilerParams(
                        needs_layout_passes=False))
    return run(row, col, xvs, xus, xe_v2u_f, xe_u2v_f)


_EPT = N_E // NW
_GBATCH = 2000
_NGB = _EPT // _GBATCH


def _puv_body(row_hbm, col_hbm, pu_hbm, pv_hbm, outu_hbm, outv_hbm,
              ridx, cidx, bufu, bufv, sem1, sem2):
    cid = lax.axis_index("c")
    sid = lax.axis_index("s")
    wid = sid * NC + cid

    def batch_body(b, carry):
        base = wid * _EPT + b * _GBATCH
        pltpu.sync_copy(row_hbm.at[pl.ds(base, _GBATCH)], ridx)
        pltpu.sync_copy(col_hbm.at[pl.ds(base, _GBATCH)], cidx)
        d1 = pltpu.async_copy(pu_hbm.at[ridx], bufu, sem1)
        d2 = pltpu.async_copy(pv_hbm.at[cidx], bufv, sem2)
        d1.wait()
        d2.wait()
        pltpu.sync_copy(bufu, outu_hbm.at[pl.ds(base, _GBATCH)])
        pltpu.sync_copy(bufv, outv_hbm.at[pl.ds(base, _GBATCH)])
        return carry

    lax.fori_loop(0, _NGB, batch_body, 0)


def _sc_puv(row, col, pu, pv):
    mesh = plsc.VectorSubcoreMesh(core_axis_name="c", subcore_axis_name="s")
    run = pl.kernel(
        _puv_body,
        out_type=[jax.ShapeDtypeStruct((N_E, 16), jnp.float32),
                  jax.ShapeDtypeStruct((N_E, 16), jnp.float32)],
        mesh=mesh,
        scratch_types=[
            pltpu.VMEM((_GBATCH,), jnp.int32),
            pltpu.VMEM((_GBATCH,), jnp.int32),
            pltpu.VMEM((_GBATCH, 16), jnp.float32),
            pltpu.VMEM((_GBATCH, 16), jnp.float32),
            pltpu.SemaphoreType.DMA,
            pltpu.SemaphoreType.DMA,
        ],
        compiler_params=pltpu.CompilerParams(use_tc_tiling_on_sc=False),
    )
    return run(row, col, pu, pv)


def _node_assemble(x_ref, s_ref, m_ref, es_ref, em_ref, cnt_ref,
                   w1, w2, w3, w4, w5, b_ref, g_ref, be_ref, out_ref):
    n = out_ref.shape[0]
    cnt = cnt_ref[...][:, :1]
    inv = 1.0 / jnp.maximum(cnt, 1.0)
    nonempty = cnt > 0.0
    mn = s_ref[...] * inv
    mx = jnp.where(nonempty, m_ref[...], 0.0)
    emn = es_ref[...] * inv
    emx = jnp.where(nonempty, em_ref[...], 0.0)
    y = (jnp.dot(x_ref[...], w1[...], preferred_element_type=jnp.float32)
         + jnp.dot(mn[0:n], w2[...], preferred_element_type=jnp.float32)
         + jnp.dot(mx[0:n], w3[...], preferred_element_type=jnp.float32)
         + jnp.dot(emn[0:n], w4[...], preferred_element_type=jnp.float32)
         + jnp.dot(emx[0:n], w5[...], preferred_element_type=jnp.float32)
         + b_ref[...])
    mu = jnp.mean(y, axis=0, keepdims=True)
    var = jnp.mean(jnp.square(y - mu), axis=0, keepdims=True)
    out_ref[...] = g_ref[...] * (y - mu) * lax.rsqrt(var + EPS) + be_ref[...]


def _node_out(x, s, m, es, em, cnt, W, b, g, be):
    w1 = W[:, 0:128].T
    w2 = W[:, 128:256].T
    w3 = W[:, 256:384].T
    w4 = W[:, 384:400].T
    w5 = W[:, 400:416].T
    n = x.shape[0]
    return pl.pallas_call(
        _node_assemble,
        out_shape=jax.ShapeDtypeStruct((n, 128), jnp.float32),
    )(x, s, m, es, em, cnt, w1, w2, w3, w4, w5,
      b.reshape(1, 128), g.reshape(1, 128), be.reshape(1, 128))


def _proj_kernel(xu_ref, xv_ref, wu_ref, wv_ref, pu_ref, pv_ref):
    pu_ref[...] = jnp.dot(xu_ref[...], wu_ref[...],
                          preferred_element_type=jnp.float32)
    pv_ref[...] = jnp.dot(xv_ref[...], wv_ref[...],
                          preferred_element_type=jnp.float32)


def _projections(xut, xvt, We):
    return pl.pallas_call(
        _proj_kernel,
        out_shape=[jax.ShapeDtypeStruct((N_U, 16), jnp.float32),
                   jax.ShapeDtypeStruct((N_V, 16), jnp.float32)],
    )(xut, xvt, We[:, 16:144].T, We[:, 144:272].T)


_EBLK = 4000


def _edge_pass1(xe_ref, pu_ref, pv_ref, bd_ref, bias_ref, ye_ref, stats_ref):
    i = pl.program_id(0)
    y = (jnp.dot(xe_ref[...], bd_ref[...], preferred_element_type=jnp.float32)
         + pu_ref[...] + pv_ref[...] + bias_ref[...])
    ye_ref[...] = y

    @pl.when(i == 0)
    def _():
        stats_ref[...] = jnp.zeros_like(stats_ref)

    s = jnp.sum(y, axis=0, keepdims=True)
    ss = jnp.sum(y * y, axis=0, keepdims=True)
    stats_ref[0:1, :] += s
    stats_ref[1:2, :] += ss


def _edge_pass2(ye_ref, scale_ref, shift_ref, out_ref):
    out_ref[...] = ye_ref[...] * scale_ref[...] + shift_ref[...]


def _edge_out(xe_f, pu_f, pv_f, We_e, be, ge, bee):
    bd = jnp.kron(jnp.eye(8, dtype=jnp.float32), We_e.T)
    bias = jnp.tile(be, 8).reshape(1, 128)
    nrows = N_E // 8
    nblk = nrows // _EBLK
    ye, stats = pl.pallas_call(
        _edge_pass1,
        grid=(nblk,),
        in_specs=[
            pl.BlockSpec((_EBLK, 128), lambda i: (i, 0)),
            pl.BlockSpec((_EBLK, 128), lambda i: (i, 0)),
            pl.BlockSpec((_EBLK, 128), lambda i: (i, 0)),
            pl.BlockSpec((128, 128), lambda i: (0, 0)),
            pl.BlockSpec((1, 128), lambda i: (0, 0)),
        ],
        out_specs=[
            pl.BlockSpec((_EBLK, 128), lambda i: (i, 0)),
            pl.BlockSpec((8, 128), lambda i: (0, 0)),
        ],
        out_shape=[
            jax.ShapeDtypeStruct((nrows, 128), jnp.float32),
            jax.ShapeDtypeStruct((8, 128), jnp.float32),
        ],
    )(xe_f, pu_f, pv_f, bd, bias)
    s16 = jnp.sum(stats[0].reshape(8, 16), axis=0)
    ss16 = jnp.sum(stats[1].reshape(8, 16), axis=0)
    mu = s16 / N_E
    var = ss16 / N_E - mu * mu
    scale = ge * lax.rsqrt(var + EPS)
    shift = bee - mu * scale
    out = pl.pallas_call(
        _edge_pass2,
        grid=(nblk,),
        in_specs=[
            pl.BlockSpec((_EBLK, 128), lambda i: (i, 0)),
            pl.BlockSpec((1, 128), lambda i: (0, 0)),
            pl.BlockSpec((1, 128), lambda i: (0, 0)),
        ],
        out_specs=pl.BlockSpec((_EBLK, 128), lambda i: (i, 0)),
        out_shape=jax.ShapeDtypeStruct((nrows, 128), jnp.float32),
    )(ye, jnp.tile(scale, 8).reshape(1, 128), jnp.tile(shift, 8).reshape(1, 128))
    return out.reshape(N_E, 16)


def kernel(xus, xut, xvs, xvt, edge_index, xe_e, xe_v2u, xe_u2v,
           Wu, bu, gu, beu, Wv, bv, gv, bev, We, be, ge, bee):
    row = jnp.asarray(edge_index[0], jnp.int32)
    col = jnp.asarray(edge_index[1], jnp.int32)

    (s_u, m_u, aux_u,
     s_v, m_v, aux_v) = _sc_segments(row, col, xvs, xus,
                                     xe_v2u.reshape(N_E // 8, 128),
                                     xe_u2v.reshape(N_E // 8, 128))

    def unaux(aux):
        es = aux[0:NPAD * 16].reshape(NPAD, 16)
        em = aux[NPAD * 16:2 * NPAD * 16].reshape(NPAD, 16)
        c = aux[2 * NPAD * 16:].reshape(NPAD, 16)
        return es, em, c

    es_u, em_u, cnt_u = unaux(aux_u)
    es_v, em_v, cnt_v = unaux(aux_v)

    out_u = _node_out(xut, s_u, m_u, es_u, em_u, cnt_u, Wu, bu, gu, beu)
    out_v = _node_out(xvt, s_v, m_v, es_v, em_v, cnt_v, Wv, bv, gv, bev)

    pu, pv = _projections(xut, xvt, We)
    pu_e, pv_e = _sc_puv(row, col, pu, pv)
    out_e = _edge_out(xe_e.reshape(N_E // 8, 128),
                      pu_e.reshape(N_E // 8, 128),
                      pv_e.reshape(N_E // 8, 128),
                      We[:, 0:16], be, ge, bee)
    return (out_u, out_v, out_e)
